# baseline (speedup 1.0000x reference)
import functools

import jax
import jax.numpy as jnp
from jax.experimental import pallas as pl
from jax.experimental.pallas import tpu as pltpu

EPS = 1e-5


def _fill_pad(pad_ref, src, H):
    pad_ref[1:H + 1, :, :] = src
    pad_ref[0:1, :, :] = src[1:2, :, :]
    pad_ref[H + 1:H + 2, :, :] = src[H - 2:H - 1, :, :]


def _conv_pass(pad_ref, w_ref, h_ref, H, W, C, sr, mask_l, mask_r):
    srw = sr * W

    def strip_body(s, carry):
        acc_sum, acc_ssq = carry
        r0 = pl.multiple_of(s * sr, sr)
        acc = jnp.zeros((srw, 3 * C), jnp.float32)
        for dy in range(3):
            slab = pad_ref[pl.ds(r0 + dy, sr), :, :].reshape(srw, C)
            acc = acc + jnp.dot(slab, w_ref[dy],
                                preferred_element_type=jnp.float32)
        c0 = acc[:, 0:C]
        c1 = acc[:, C:2 * C]
        c2 = acc[:, 2 * C:3 * C]
        prev0 = pltpu.roll(c0, 1, 0)
        next0 = pltpu.roll(c0, srw - 1, 0)
        prev2 = pltpu.roll(c2, 1, 0)
        next2 = pltpu.roll(c2, srw - 1, 0)
        y = (c1 + jnp.where(mask_l, next0, prev0)
             + jnp.where(mask_r, prev2, next2))
        row0 = pl.multiple_of(r0 * W, srw)
        h_ref[pl.ds(row0, srw), :] = y.astype(h_ref.dtype)
        return (acc_sum + jnp.sum(y, axis=0, keepdims=True),
                acc_ssq + jnp.sum(y * y, axis=0, keepdims=True))

    zero = jnp.zeros((1, C), jnp.float32)
    return jax.lax.fori_loop(0, H // sr, strip_body, (zero, zero))


def _rb_kernel(x_ref, w1_ref, w2_ref, o_ref, pad_ref, h_ref, *, sr):
    H, W, C = x_ref.shape
    inv_n = 1.0 / (H * W)
    srw = sr * W

    col = jax.lax.broadcasted_iota(jnp.int32, (srw, C), 0) % W
    mask_l = col == 0
    mask_r = col == (W - 1)

    _fill_pad(pad_ref, x_ref[...].astype(pad_ref.dtype), H)
    s1, ss1 = _conv_pass(pad_ref, w1_ref, h_ref, H, W, C, sr, mask_l, mask_r)
    mean1 = s1 * inv_n
    var1 = jnp.maximum(ss1 * inv_n - mean1 * mean1, 0.0)
    scale1 = jax.lax.rsqrt(var1 + EPS)

    h1 = jnp.maximum((h_ref[...].astype(jnp.float32) - mean1) * scale1, 0.0)
    _fill_pad(pad_ref, h1.reshape(H, W, C).astype(pad_ref.dtype), H)

    s2, ss2 = _conv_pass(pad_ref, w2_ref, h_ref, H, W, C, sr, mask_l, mask_r)
    mean2 = s2 * inv_n
    var2 = jnp.maximum(ss2 * inv_n - mean2 * mean2, 0.0)
    scale2 = jax.lax.rsqrt(var2 + EPS)

    h2 = (h_ref[...].astype(jnp.float32) - mean2) * scale2
    o_ref[...] = (x_ref[...].astype(jnp.float32)
                  + h2.reshape(H, W, C)).astype(o_ref.dtype)


def _pick_strip_rows(H, W, C):
    max_rows = max(1, (192 * 1024) // (W * 3 * C * 4))
    sr = min(H, max_rows)
    while H % sr:
        sr -= 1
    return sr


def kernel(x, w1, b1, w2, b2):
    del b1, b2
    N, C, H, W = x.shape
    if H < 2 or W < 2:
        raise ValueError("reflect padding of 1 requires H >= 2 and W >= 2")

    Cp = max(128, -(-C // 128) * 128)
    xt = jnp.transpose(x, (0, 2, 3, 1))

    def prep(w):
        t = jnp.transpose(w, (2, 1, 3, 0))
        if Cp != C:
            t = jnp.pad(t, ((0, 0), (0, Cp - C), (0, 0), (0, Cp - C)))
        return t.reshape(3, Cp, 3 * Cp).astype(jnp.bfloat16)

    w1a = prep(w1)
    w2a = prep(w2)
    if Cp != C:
        xt = jnp.pad(xt, ((0, 0), (0, 0), (0, 0), (0, Cp - C)))

    sr = _pick_strip_rows(H, W, Cp)

    out = pl.pallas_call(
        functools.partial(_rb_kernel, sr=sr),
        out_shape=jax.ShapeDtypeStruct((N, H, W, Cp), x.dtype),
        grid=(N,),
        in_specs=[
            pl.BlockSpec((None, H, W, Cp), lambda n: (n, 0, 0, 0)),
            pl.BlockSpec((3, Cp, 3 * Cp), lambda n: (0, 0, 0)),
            pl.BlockSpec((3, Cp, 3 * Cp), lambda n: (0, 0, 0)),
        ],
        out_specs=pl.BlockSpec((None, H, W, Cp), lambda n: (n, 0, 0, 0)),
        scratch_shapes=[
            pltpu.VMEM((H + 2, W, Cp), jnp.bfloat16),
            pltpu.VMEM((H * W, Cp), jnp.bfloat16),
        ],
        compiler_params=pltpu.CompilerParams(
            dimension_semantics=("parallel",),
            vmem_limit_bytes=48 * 1024 * 1024),
    )(xt, w1a, w2a)

    if Cp != C:
        out = out[..., :C]
    return jnp.transpose(out, (0, 3, 1, 2))


# speedup vs baseline: 1.5510x; 1.5510x over previous
import functools

import jax
import jax.numpy as jnp
from jax.experimental import pallas as pl
from jax.experimental.pallas import tpu as pltpu

EPS = 1e-5


def _rb_kernel(x_ref, w1_ref, w2_ref, o_ref, x3_ref, y_ref, h_ref, *, H, W):
    C = x_ref.shape[0]
    HW = H * W
    inv_n = 1.0 / HW

    lane = jax.lax.broadcasted_iota(jnp.int32, (1, HW), 1)
    col = lane % W
    mask_l = col == 0
    mask_r = col == (W - 1)
    mask_top = lane < W
    mask_bot = lane >= (H - 1) * W

    def build_x3(src):
        rp = pltpu.roll(src, W, 1)
        rm = pltpu.roll(src, HW - W, 1)
        x3_ref[0:C, :] = jnp.where(mask_top, rm, rp)
        x3_ref[C:2 * C, :] = src
        x3_ref[2 * C:3 * C, :] = jnp.where(mask_bot, rp, rm)

    def conv(w_ref):
        y_ref[...] = jnp.dot(w_ref[...], x3_ref[...],
                             preferred_element_type=jnp.float32)
        c0 = y_ref[0:C, :]
        c1 = y_ref[C:2 * C, :]
        c2 = y_ref[2 * C:3 * C, :]
        p0 = pltpu.roll(c0, 1, 1)
        m0 = pltpu.roll(c0, HW - 1, 1)
        p2 = pltpu.roll(c2, 1, 1)
        m2 = pltpu.roll(c2, HW - 1, 1)
        y = (c1 + jnp.where(mask_l, m0, p0) + jnp.where(mask_r, p2, m2))
        h_ref[...] = y.astype(h_ref.dtype)
        s = jnp.sum(y, axis=1, keepdims=True)
        ss = jnp.sum(y * y, axis=1, keepdims=True)
        mean = s * inv_n
        var = jnp.maximum(ss * inv_n - mean * mean, 0.0)
        return mean, jax.lax.rsqrt(var + EPS)

    build_x3(x_ref[...].astype(x3_ref.dtype))
    mean1, scale1 = conv(w1_ref)

    h1 = jnp.maximum((h_ref[...].astype(jnp.float32) - mean1) * scale1, 0.0)
    build_x3(h1.astype(x3_ref.dtype))

    mean2, scale2 = conv(w2_ref)

    h2 = (h_ref[...].astype(jnp.float32) - mean2) * scale2
    o_ref[...] = (x_ref[...].astype(jnp.float32) + h2).astype(o_ref.dtype)


def kernel(x, w1, b1, w2, b2):
    del b1, b2
    N, C, H, W = x.shape
    if H < 2 or W < 2:
        raise ValueError("reflect padding of 1 requires H >= 2 and W >= 2")

    xf = x.reshape(N, C, H * W)

    def prep(w):
        t = jnp.transpose(w, (3, 0, 2, 1))
        return t.reshape(3 * C, 3 * C).astype(jnp.bfloat16)

    out = pl.pallas_call(
        functools.partial(_rb_kernel, H=H, W=W),
        out_shape=jax.ShapeDtypeStruct((N, C, H * W), x.dtype),
        grid=(N,),
        in_specs=[
            pl.BlockSpec((None, C, H * W), lambda n: (n, 0, 0)),
            pl.BlockSpec((3 * C, 3 * C), lambda n: (0, 0)),
            pl.BlockSpec((3 * C, 3 * C), lambda n: (0, 0)),
        ],
        out_specs=pl.BlockSpec((None, C, H * W), lambda n: (n, 0, 0)),
        scratch_shapes=[
            pltpu.VMEM((3 * C, H * W), jnp.bfloat16),
            pltpu.VMEM((3 * C, H * W), jnp.float32),
            pltpu.VMEM((C, H * W), jnp.bfloat16),
        ],
        compiler_params=pltpu.CompilerParams(
            dimension_semantics=("parallel",),
            vmem_limit_bytes=48 * 1024 * 1024),
    )(xf, prep(w1), prep(w2))

    return out.reshape(N, C, H, W)


# speedup vs baseline: 1.9731x; 1.2721x over previous
import functools

import jax
import jax.numpy as jnp
from jax.experimental import pallas as pl
from jax.experimental.pallas import tpu as pltpu

EPS = 1e-5


def _rb_kernel(x_ref, w1_ref, w2_ref, o_ref, x3_ref, y_ref, h_ref, *, H, W):
    C = x_ref.shape[0]
    HW = H * W
    inv_n = 1.0 / HW

    lane = jax.lax.broadcasted_iota(jnp.int32, (1, HW), 1)
    col = lane % W
    mask_l = col == 0
    mask_r = col == (W - 1)
    mask_top = lane < W
    mask_bot = lane >= (H - 1) * W

    def build_x3(src):
        rp = pltpu.roll(src, W, 1)
        rm = pltpu.roll(src, HW - W, 1)
        x3_ref[0:C, :] = jnp.where(mask_top, rm, rp)
        x3_ref[C:2 * C, :] = src
        x3_ref[2 * C:3 * C, :] = jnp.where(mask_bot, rp, rm)

    def conv(w_ref):
        y_ref[...] = jnp.dot(w_ref[...], x3_ref[...],
                             preferred_element_type=jnp.float32
                             ).astype(y_ref.dtype)
        c0 = y_ref[0:C, :]
        c1 = y_ref[C:2 * C, :]
        c2 = y_ref[2 * C:3 * C, :]
        p0 = pltpu.roll(c0, 1, 1)
        m0 = pltpu.roll(c0, HW - 1, 1)
        p2 = pltpu.roll(c2, 1, 1)
        m2 = pltpu.roll(c2, HW - 1, 1)
        y = (c1.astype(jnp.float32)
             + jnp.where(mask_l, m0, p0).astype(jnp.float32)
             + jnp.where(mask_r, p2, m2).astype(jnp.float32))
        h_ref[...] = y.astype(h_ref.dtype)
        s = jnp.sum(y, axis=1, keepdims=True)
        ss = jnp.sum(y * y, axis=1, keepdims=True)
        mean = s * inv_n
        var = jnp.maximum(ss * inv_n - mean * mean, 0.0)
        return mean, jax.lax.rsqrt(var + EPS)

    build_x3(x_ref[...].astype(x3_ref.dtype))
    mean1, scale1 = conv(w1_ref)

    h1 = jnp.maximum((h_ref[...].astype(jnp.float32) - mean1) * scale1, 0.0)
    build_x3(h1.astype(x3_ref.dtype))

    mean2, scale2 = conv(w2_ref)

    h2 = (h_ref[...].astype(jnp.float32) - mean2) * scale2
    o_ref[...] = (x_ref[...].astype(jnp.float32) + h2).astype(o_ref.dtype)


def kernel(x, w1, b1, w2, b2):
    del b1, b2
    N, C, H, W = x.shape
    if H < 2 or W < 2:
        raise ValueError("reflect padding of 1 requires H >= 2 and W >= 2")

    xf = x.reshape(N, C, H * W)

    def prep(w):
        t = jnp.transpose(w, (3, 0, 2, 1))
        return t.reshape(3 * C, 3 * C).astype(jnp.bfloat16)

    out = pl.pallas_call(
        functools.partial(_rb_kernel, H=H, W=W),
        out_shape=jax.ShapeDtypeStruct((N, C, H * W), x.dtype),
        grid=(N,),
        in_specs=[
            pl.BlockSpec((None, C, H * W), lambda n: (n, 0, 0)),
            pl.BlockSpec((3 * C, 3 * C), lambda n: (0, 0)),
            pl.BlockSpec((3 * C, 3 * C), lambda n: (0, 0)),
        ],
        out_specs=pl.BlockSpec((None, C, H * W), lambda n: (n, 0, 0)),
        scratch_shapes=[
            pltpu.VMEM((3 * C, H * W), jnp.bfloat16),
            pltpu.VMEM((3 * C, H * W), jnp.bfloat16),
            pltpu.VMEM((C, H * W), jnp.bfloat16),
        ],
        compiler_params=pltpu.CompilerParams(
            dimension_semantics=("parallel",),
            vmem_limit_bytes=48 * 1024 * 1024),
    )(xf, prep(w1), prep(w2))

    return out.reshape(N, C, H, W)
